# baseline (speedup 1.0000x reference)
import functools

import jax
import jax.numpy as jnp
from jax import lax
from jax.experimental import pallas as pl
from jax.experimental.pallas import tpu as pltpu
from jax.experimental.pallas import tpu_sc as plsc

_N = 10000
_NP = 10240
_C = 128
_TW = 2 * _C
_K = 64
_R2 = 0.09 * 0.09
_BIGF = 1e30

_B1 = 256


def _topk_body(pos8_ref, posT8_ref, x_ref, pn_ref, posW_ref,
               idx_ref, vf_ref, t_ref, d2s):
    t_ref[:, 0:_C] = x_ref[...]
    t_ref[:, _C:_TW] = jnp.dot(pn_ref[...], posW_ref[...],
                               preferred_element_type=jnp.float32)
    pos_blk = pos8_ref[...]
    posT = posT8_ref[...]
    dot = jnp.dot(pos_blk, posT, preferred_element_type=jnp.float32)
    sq_i = jnp.sum(pos_blk * pos_blk, axis=1, keepdims=True)
    sq_j = jnp.sum(posT * posT, axis=0, keepdims=True)
    d2 = sq_i + sq_j - 2.0 * dot
    inrad = d2 <= _R2
    cnt = jnp.sum(inrad.astype(jnp.float32), axis=1, keepdims=True)
    col = lax.broadcasted_iota(jnp.int32, (_B1, _K), 1)
    vf_ref[...] = jnp.where(col.astype(jnp.float32) < cnt, 1.0, 0.0)
    lane = lax.broadcasted_iota(jnp.int32, (_B1, _NP), 1)
    dbits = lax.bitcast_convert_type(jnp.maximum(d2, 0.0), jnp.int32)
    bigk = jnp.int32(0x7F000000)
    d2s[...] = jnp.where(inrad, (dbits & jnp.int32(~0x3FFF)) | lane, bigk)
    idx_ref[...] = jnp.zeros((_B1, _K), jnp.int32)

    def cond(carry):
        k, cont = carry
        return cont

    def body(carry):
        k, _ = carry
        keys = d2s[...]
        mn = jnp.min(keys, axis=1, keepdims=True)
        d2s[...] = jnp.where(keys == mn, bigk, keys)
        idx_ref[...] = jnp.where(col == k, mn & jnp.int32(0x3FFF),
                                 idx_ref[...])
        alive = jnp.min(mn) < bigk
        return k + 1, jnp.logical_and(k + 1 < _K, alive)

    lax.while_loop(cond, body, (jnp.int32(0), jnp.bool_(True)))


def _topk(pos8, posT8, xp, pn16, posW16):
    return pl.pallas_call(
        _topk_body,
        grid=(_NP // _B1,),
        in_specs=[
            pl.BlockSpec((_B1, 8), lambda b: (b, 0)),
            pl.BlockSpec((8, _NP), lambda b: (0, 0)),
            pl.BlockSpec((_B1, _C), lambda b: (b, 0)),
            pl.BlockSpec((_B1, 16), lambda b: (b, 0)),
            pl.BlockSpec((16, _C), lambda b: (0, 0)),
        ],
        out_specs=[
            pl.BlockSpec((_B1, _K), lambda b: (b, 0)),
            pl.BlockSpec((_B1, _K), lambda b: (b, 0)),
            pl.BlockSpec((_B1, _TW), lambda b: (b, 0)),
        ],
        out_shape=[
            jax.ShapeDtypeStruct((_NP, _K), jnp.int32),
            jax.ShapeDtypeStruct((_NP, _K), jnp.float32),
            jax.ShapeDtypeStruct((_NP, _TW), jnp.float32),
        ],
        scratch_shapes=[pltpu.VMEM((_B1, _NP), jnp.int32)],
    )(pos8, posT8, xp, pn16, posW16)


_E = _NP * _K
_CH = 128
_CI = 8
_HF = 2


def _sc_gather(tbl, idx2d):
    info = plsc.get_sparse_core_info()
    nw = info.num_cores * info.num_subcores
    rows_w = _E // nw
    outer = rows_w // (_CH * _CI)
    mesh = plsc.VectorSubcoreMesh(core_axis_name="c", subcore_axis_name="s")

    @functools.partial(
        pl.kernel, mesh=mesh,
        out_type=jax.ShapeDtypeStruct((_E, _TW), jnp.float32),
        scratch_types=[
            pltpu.VMEM((_CI, _CH), jnp.int32),
            pltpu.VMEM((_HF * _CH, _TW), jnp.float32),
            pltpu.SemaphoreType.DMA,
        ],
    )
    def k(t_hbm, idx_hbm, out_hbm, idx_v, r1, s1):
        wid = lax.axis_index("s") * info.num_cores + lax.axis_index("c")
        base = wid * rows_w

        def step(g, carry):
            row0 = pl.multiple_of(base + g * (_CH * _CI), _CH * _CI)
            irow = pl.multiple_of(row0 // _CH, _CI)
            pltpu.sync_copy(idx_hbm.at[pl.ds(irow, _CI), :], idx_v)
            for half in range(_CI // _HF):
                d1 = [pltpu.async_copy(t_hbm.at[idx_v.at[half * _HF + j]],
                                       r1.at[pl.ds(j * _CH, _CH), :], s1)
                      for j in range(_HF)]
                for d in d1:
                    d.wait()
                o0 = pl.multiple_of(row0 + half * (_HF * _CH), _HF * _CH)
                pltpu.sync_copy(r1, out_hbm.at[pl.ds(o0, _HF * _CH), :])
            return carry

        lax.fori_loop(0, outer, step, None)

    return k(tbl, idx2d)


_B3 = 64


def _edge_body(t_ref, g_ref, vf_ref,
               linW_ref, srcW_ref, dstW_ref, posb_ref,
               attW_ref, attb_ref, out_ref):
    x_d = t_ref[:, 0:_C]
    q_i = t_ref[:, _C:_TW]
    a_dst = jnp.dot(x_d, dstW_ref[...],
                    preferred_element_type=jnp.float32)
    x_e = g_ref[:, 0:_C]
    q_e = g_ref[:, _C:_TW]
    v_e = jnp.dot(x_e, linW_ref[...], preferred_element_type=jnp.float32)
    as_e = jnp.dot(x_e, srcW_ref[...], preferred_element_type=jnp.float32)
    pb = posb_ref[...].reshape(1, 1, _C)
    delta = jax.nn.relu(q_i[:, None, :] - q_e.reshape(_B3, _K, _C) + pb)
    h = a_dst[:, None, :] - as_e.reshape(_B3, _K, _C) + delta
    alpha = jnp.dot(h.reshape(_B3 * _K, _C), attW_ref[...],
                    preferred_element_type=jnp.float32)
    alpha = jax.nn.relu(alpha + attb_ref[...].reshape(1, _C))
    r0 = lax.broadcasted_iota(jnp.int32, (_B3 * _K, _B3), 0)
    c0 = lax.broadcasted_iota(jnp.int32, (_B3 * _K, _B3), 1)
    sel_dst = (r0 // _K == c0).astype(jnp.float32)
    r1 = lax.broadcasted_iota(jnp.int32, (_B3 * _K, _K), 0)
    c1 = lax.broadcasted_iota(jnp.int32, (_B3 * _K, _K), 1)
    sel_k = (r1 % _K == c1).astype(jnp.float32)
    vf_rows = jnp.dot(sel_dst, vf_ref[...],
                      preferred_element_type=jnp.float32)
    vf_e = jnp.sum(vf_rows * sel_k, axis=1, keepdims=True)
    penalty = (vf_e - 1.0) * _BIGF
    am = (alpha + jnp.broadcast_to(penalty, (_B3 * _K, _C))).reshape(
        _B3, _K, _C)
    mx = jnp.max(am, axis=1, keepdims=True)
    e = jnp.exp(am - mx)
    s = jnp.sum(e, axis=1, keepdims=True)
    w = e / s
    msg = w * (v_e.reshape(_B3, _K, _C) + delta)
    out_ref[...] = jnp.sum(msg, axis=1)


def _edge(tbl, g, vf, linW, srcW, dstW, posb, attW, attb):
    full = lambda r, c: pl.BlockSpec((r, c), lambda b: (0, 0))
    return pl.pallas_call(
        _edge_body,
        grid=(_NP // _B3,),
        in_specs=[
            pl.BlockSpec((_B3, _TW), lambda b: (b, 0)),
            pl.BlockSpec((_B3 * _K, _TW), lambda b: (b, 0)),
            pl.BlockSpec((_B3, _K), lambda b: (b, 0)),
            full(_C, _C), full(_C, _C), full(_C, _C),
            full(1, _C), full(_C, _C), full(1, _C),
        ],
        out_specs=pl.BlockSpec((_B3, _C), lambda b: (b, 0)),
        out_shape=jax.ShapeDtypeStruct((_NP, _C), jnp.float32),
    )(tbl, g, vf, linW, srcW, dstW, posb, attW, attb)


def kernel(x, pos, normal, batch, lin_W, lin_src_W, lin_dst_W, pos_W, pos_b,
           pos_bn_w, pos_bn_b, attn_W, attn_b, attn_bn_w, attn_bn_b):
    pad = _NP - _N
    posp = jnp.pad(pos, ((0, pad), (0, 5)))
    rowpad = (jnp.arange(_NP) >= _N)[:, None]
    colxyz = (jnp.arange(8) < 3)[None, :]
    posp = posp + jnp.where(rowpad & colxyz, 1e3, 0.0).astype(jnp.float32)

    xp = jnp.pad(x, ((0, pad), (0, 0)))
    pn16 = jnp.pad(jnp.concatenate([pos, normal], axis=1),
                   ((0, pad), (0, 10)))
    posW16 = jnp.pad(pos_W, ((0, 10), (0, 0))) * pos_bn_w[None, :]
    posb = (pos_b * pos_bn_w + pos_bn_b).reshape(1, _C)
    attW = attn_W * attn_bn_w[None, :]
    attb = (attn_b * attn_bn_w + attn_bn_b).reshape(1, _C)

    idx, vf, tbl = _topk(posp, posp.T, xp, pn16, posW16)
    g = _sc_gather(tbl, idx.reshape(_E // _CH, _CH))
    out = _edge(tbl, g, vf, lin_W, lin_src_W, lin_dst_W, posb, attW, attb)
    return out[:_N]


# speedup vs baseline: 2.0747x; 2.0747x over previous
import functools

import jax
import jax.numpy as jnp
from jax import lax
from jax.experimental import pallas as pl
from jax.experimental.pallas import tpu as pltpu
from jax.experimental.pallas import tpu_sc as plsc

_N = 10000
_NP = 10240
_C = 128
_TW = 2 * _C
_K = 64
_R2 = 0.09 * 0.09
_BIGF = 1e30

_B1 = 256


def _topk_body(pos8_ref, posT8_ref, x_ref, pn_ref, posW_ref,
               idx_ref, vf_ref, t_ref, d2s):
    t_ref[:, 0:_C] = x_ref[...]
    t_ref[:, _C:_TW] = jnp.dot(pn_ref[...], posW_ref[...],
                               preferred_element_type=jnp.float32)
    pos_blk = pos8_ref[...]
    posT = posT8_ref[...]
    dot = jnp.dot(pos_blk, posT, preferred_element_type=jnp.float32)
    sq_i = jnp.sum(pos_blk * pos_blk, axis=1, keepdims=True)
    sq_j = jnp.sum(posT * posT, axis=0, keepdims=True)
    d2s[...] = sq_i + sq_j - 2.0 * dot
    lane = lax.broadcasted_iota(jnp.int32, (_B1, _NP), 1)
    col = lax.broadcasted_iota(jnp.int32, (_B1, _K), 1)

    def body(k, carry):
        d2 = d2s[...]
        mn = jnp.min(d2, axis=1, keepdims=True)
        sel = jnp.min(jnp.where(d2 == mn, lane, jnp.int32(2**30)),
                      axis=1, keepdims=True)
        d2s[...] = jnp.where(lane == sel, _BIGF, d2)
        cm = col == k
        idx_ref[...] = jnp.where(cm, sel, idx_ref[...])
        vf = jnp.where(mn <= _R2, 1.0, 0.0)
        vf_ref[...] = jnp.where(cm, vf, vf_ref[...])
        return carry

    idx_ref[...] = jnp.zeros((_B1, _K), jnp.int32)
    vf_ref[...] = jnp.zeros((_B1, _K), jnp.float32)
    lax.fori_loop(0, _K, body, None)


def _topk(pos8, posT8, xp, pn16, posW16):
    return pl.pallas_call(
        _topk_body,
        grid=(_NP // _B1,),
        in_specs=[
            pl.BlockSpec((_B1, 8), lambda b: (b, 0)),
            pl.BlockSpec((8, _NP), lambda b: (0, 0)),
            pl.BlockSpec((_B1, _C), lambda b: (b, 0)),
            pl.BlockSpec((_B1, 16), lambda b: (b, 0)),
            pl.BlockSpec((16, _C), lambda b: (0, 0)),
        ],
        out_specs=[
            pl.BlockSpec((_B1, _K), lambda b: (b, 0)),
            pl.BlockSpec((_B1, _K), lambda b: (b, 0)),
            pl.BlockSpec((_B1, _TW), lambda b: (b, 0)),
        ],
        out_shape=[
            jax.ShapeDtypeStruct((_NP, _K), jnp.int32),
            jax.ShapeDtypeStruct((_NP, _K), jnp.float32),
            jax.ShapeDtypeStruct((_NP, _TW), jnp.float32),
        ],
        scratch_shapes=[pltpu.VMEM((_B1, _NP), jnp.float32)],
    )(pos8, posT8, xp, pn16, posW16)


_E = _NP * _K
_CH = 128
_CI = 8
_HF = 2


def _sc_gather(tbl, idx2d):
    info = plsc.get_sparse_core_info()
    nw = info.num_cores * info.num_subcores
    rows_w = _E // nw
    outer = rows_w // (_CH * _CI)
    mesh = plsc.VectorSubcoreMesh(core_axis_name="c", subcore_axis_name="s")

    @functools.partial(
        pl.kernel, mesh=mesh,
        out_type=jax.ShapeDtypeStruct((_E, _TW), jnp.float32),
        scratch_types=[
            pltpu.VMEM((_CI, _CH), jnp.int32),
            pltpu.VMEM((_HF * _CH, _TW), jnp.float32),
            pltpu.SemaphoreType.DMA,
        ],
    )
    def k(t_hbm, idx_hbm, out_hbm, idx_v, r1, s1):
        wid = lax.axis_index("s") * info.num_cores + lax.axis_index("c")
        base = wid * rows_w

        def step(g, carry):
            row0 = pl.multiple_of(base + g * (_CH * _CI), _CH * _CI)
            irow = pl.multiple_of(row0 // _CH, _CI)
            pltpu.sync_copy(idx_hbm.at[pl.ds(irow, _CI), :], idx_v)
            for half in range(_CI // _HF):
                d1 = [pltpu.async_copy(t_hbm.at[idx_v.at[half * _HF + j]],
                                       r1.at[pl.ds(j * _CH, _CH), :], s1)
                      for j in range(_HF)]
                for d in d1:
                    d.wait()
                o0 = pl.multiple_of(row0 + half * (_HF * _CH), _HF * _CH)
                pltpu.sync_copy(r1, out_hbm.at[pl.ds(o0, _HF * _CH), :])
            return carry

        lax.fori_loop(0, outer, step, None)

    return k(tbl, idx2d)


_B3 = 64


def _edge_body(t_ref, g_ref, vf_ref,
               linW_ref, srcW_ref, dstW_ref, posb_ref,
               attW_ref, attb_ref, out_ref):
    x_d = t_ref[:, 0:_C]
    q_i = t_ref[:, _C:_TW]
    a_dst = jnp.dot(x_d, dstW_ref[...],
                    preferred_element_type=jnp.float32)
    x_e = g_ref[:, 0:_C]
    q_e = g_ref[:, _C:_TW]
    v_e = jnp.dot(x_e, linW_ref[...], preferred_element_type=jnp.float32)
    as_e = jnp.dot(x_e, srcW_ref[...], preferred_element_type=jnp.float32)
    pb = posb_ref[...].reshape(1, 1, _C)
    delta = jax.nn.relu(q_i[:, None, :] - q_e.reshape(_B3, _K, _C) + pb)
    h = a_dst[:, None, :] - as_e.reshape(_B3, _K, _C) + delta
    alpha = jnp.dot(h.reshape(_B3 * _K, _C), attW_ref[...],
                    preferred_element_type=jnp.float32)
    alpha = jax.nn.relu(alpha + attb_ref[...].reshape(1, _C))
    r0 = lax.broadcasted_iota(jnp.int32, (_B3 * _K, _B3), 0)
    c0 = lax.broadcasted_iota(jnp.int32, (_B3 * _K, _B3), 1)
    sel_dst = (r0 // _K == c0).astype(jnp.float32)
    r1 = lax.broadcasted_iota(jnp.int32, (_B3 * _K, _K), 0)
    c1 = lax.broadcasted_iota(jnp.int32, (_B3 * _K, _K), 1)
    sel_k = (r1 % _K == c1).astype(jnp.float32)
    vf_rows = jnp.dot(sel_dst, vf_ref[...],
                      preferred_element_type=jnp.float32)
    vf_e = jnp.sum(vf_rows * sel_k, axis=1, keepdims=True)
    penalty = (vf_e - 1.0) * _BIGF
    am = (alpha + jnp.broadcast_to(penalty, (_B3 * _K, _C))).reshape(
        _B3, _K, _C)
    mx = jnp.max(am, axis=1, keepdims=True)
    e = jnp.exp(am - mx)
    s = jnp.sum(e, axis=1, keepdims=True)
    w = e / s
    msg = w * (v_e.reshape(_B3, _K, _C) + delta)
    out_ref[...] = jnp.sum(msg, axis=1)


def _edge(tbl, g, vf, linW, srcW, dstW, posb, attW, attb):
    full = lambda r, c: pl.BlockSpec((r, c), lambda b: (0, 0))
    return pl.pallas_call(
        _edge_body,
        grid=(_NP // _B3,),
        in_specs=[
            pl.BlockSpec((_B3, _TW), lambda b: (b, 0)),
            pl.BlockSpec((_B3 * _K, _TW), lambda b: (b, 0)),
            pl.BlockSpec((_B3, _K), lambda b: (b, 0)),
            full(_C, _C), full(_C, _C), full(_C, _C),
            full(1, _C), full(_C, _C), full(1, _C),
        ],
        out_specs=pl.BlockSpec((_B3, _C), lambda b: (b, 0)),
        out_shape=jax.ShapeDtypeStruct((_NP, _C), jnp.float32),
    )(tbl, g, vf, linW, srcW, dstW, posb, attW, attb)


def kernel(x, pos, normal, batch, lin_W, lin_src_W, lin_dst_W, pos_W, pos_b,
           pos_bn_w, pos_bn_b, attn_W, attn_b, attn_bn_w, attn_bn_b):
    pad = _NP - _N
    posp = jnp.pad(pos, ((0, pad), (0, 5)))
    rowpad = (jnp.arange(_NP) >= _N)[:, None]
    colxyz = (jnp.arange(8) < 3)[None, :]
    posp = posp + jnp.where(rowpad & colxyz, 1e3, 0.0).astype(jnp.float32)

    xp = jnp.pad(x, ((0, pad), (0, 0)))
    pn16 = jnp.pad(jnp.concatenate([pos, normal], axis=1),
                   ((0, pad), (0, 10)))
    posW16 = jnp.pad(pos_W, ((0, 10), (0, 0))) * pos_bn_w[None, :]
    posb = (pos_b * pos_bn_w + pos_bn_b).reshape(1, _C)
    attW = attn_W * attn_bn_w[None, :]
    attb = (attn_b * attn_bn_w + attn_bn_b).reshape(1, _C)

    idx, vf, tbl = _topk(posp, posp.T, xp, pn16, posW16)
    g = _sc_gather(tbl, idx.reshape(_E // _CH, _CH))
    out = _edge(tbl, g, vf, lin_W, lin_src_W, lin_dst_W, posb, attW, attb)
    return out[:_N]
